# baseline (speedup 1.0000x reference)
import functools
import jax
import jax.numpy as jnp
import numpy as np
from jax import lax
from jax.experimental import pallas as pl
from jax.experimental.pallas import tpu as pltpu
from jax.experimental.pallas import tpu_sc as plsc

EPS = 1e-5
K = 20
B, N = 2, 2048
CIN = 1216
BLKA = 256
NBLKA = N // BLKA
BLK = 128
NBLK = N // BLK
BN = B * N

NC, NS, L = 2, 16, 16
NW = NC * NS
PTS_W = N // NW
P_SUB = 4
ROWS = P_SUB * K
NSUB = PTS_W // P_SUB


def _ka_body(x_ref, pb_ref, pf_ref, w1_ref, s1_ref, c1_ref, wa_ref, wc_ref,
             s2_ref, s2p_ref, c2_ref, ha_ref, cb_ref, idx_ref):
    xb = x_ref[0]
    h1 = lax.dot_general(xb, w1_ref[...], (((0,), (1,)), ((), ())),
                         preferred_element_type=jnp.float32)
    h1 = h1 * s1_ref[...] + c1_ref[...]
    h1 = jnp.where(h1 >= 0, h1, 0.2 * h1)
    s2 = s2_ref[...]
    ha = lax.dot_general(h1, wa_ref[...], (((1,), (1,)), ((), ())),
                         preferred_element_type=jnp.float32) * s2
    def skey(v):
        b = lax.bitcast_convert_type(v.astype(jnp.bfloat16),
                                     jnp.uint16).astype(jnp.uint32)
        return jnp.where(b >= 0x8000, 0xFFFF - b, b + 0x8000)

    lo = skey(ha[:, :128])
    hi = skey(ha[:, 128:])
    ha_ref[...] = (lo | (hi << 16)).astype(jnp.int32)
    cb_ref[...] = lax.dot_general(h1, wc_ref[...], (((1,), (1,)), ((), ())),
                                  preferred_element_type=jnp.float32) * s2p_ref[...] + c2_ref[...]

    pf = pf_ref[0]
    pb = pb_ref[0]
    ppf = jnp.sum(pf * pf, axis=0)
    ppb = jnp.sum(pb * pb, axis=0)
    G = lax.dot_general(pb, pf, (((0,), (0,)), ((), ())),
                        preferred_element_type=jnp.float32)
    nd = 2.0 * G - ppb[:, None] - ppf[None, :]
    iota_f = lax.broadcasted_iota(jnp.int32, (BLKA, N), 1).astype(jnp.float32)
    for j in range(K):
        m = jnp.max(nd, axis=1, keepdims=True)
        eqm = nd == m
        cand = jnp.where(eqm, iota_f, 4096.0)
        amin = jnp.min(cand, axis=1, keepdims=True)
        idx_ref[:, j] = amin[:, 0].astype(jnp.int32)
        nd = jnp.where(eqm, -jnp.inf, nd)


def _stage_a(x, p, w1, s1, c1, wa, wc, s2, s2p, c2, b):
    grid = (NBLKA,)
    return pl.pallas_call(
        _ka_body,
        grid=grid,
        in_specs=[
            pl.BlockSpec((1, CIN, BLKA), lambda g, _b=b: (_b, 0, g)),
            pl.BlockSpec((1, 3, BLKA), lambda g, _b=b: (_b, 0, g)),
            pl.BlockSpec((1, 3, N), lambda g, _b=b: (_b, 0, 0)),
            pl.BlockSpec((256, CIN), lambda g: (0, 0)),
            pl.BlockSpec((1, 256), lambda g: (0, 0)),
            pl.BlockSpec((1, 256), lambda g: (0, 0)),
            pl.BlockSpec((256, 256), lambda g: (0, 0)),
            pl.BlockSpec((256, 256), lambda g: (0, 0)),
            pl.BlockSpec((1, 256), lambda g: (0, 0)),
            pl.BlockSpec((1, 256), lambda g: (0, 0)),
            pl.BlockSpec((1, 256), lambda g: (0, 0)),
        ],
        out_specs=[
            pl.BlockSpec((BLKA, 128), lambda g: (g, 0)),
            pl.BlockSpec((BLKA, 256), lambda g: (g, 0)),
            pl.BlockSpec((BLKA, K), lambda g: (g, 0)),
        ],
        out_shape=[
            jax.ShapeDtypeStruct((N, 128), jnp.int32),
            jax.ShapeDtypeStruct((N, 256), jnp.float32),
            jax.ShapeDtypeStruct((N, K), jnp.int32),
        ],
    )(x, p, p, w1, s1, c1, wa, wc, s2, s2p, c2)


def _kb_body(ha_hbm, idx_hbm, out_hbm, idx_v, rows0, rows1, outv,
             sem0, sem1):
    wid = lax.axis_index("s") * NC + lax.axis_index("c")
    base_pt = wid * PTS_W
    pltpu.sync_copy(idx_hbm.at[wid], idx_v)

    rows = (rows0, rows1)
    sems = (sem0, sem1)

    def gather(c, slot):
        return pltpu.async_copy(ha_hbm.at[idx_v.at[c]], rows[slot], sems[slot])

    def reduce_chunk(c, slot):
        buf = rows[slot]

        def ch_body(ch, _):
            cs = ch * L
            for pt in range(P_SUB):
                r0 = pt * K
                v = buf[r0, pl.ds(cs, L)]
                alo = v & 0xFFFF
                ahi = lax.shift_right_logical(v, 16)
                for j in range(1, K):
                    v = buf[r0 + j, pl.ds(cs, L)]
                    alo = jnp.maximum(alo, v & 0xFFFF)
                    ahi = jnp.maximum(ahi, lax.shift_right_logical(v, 16))
                outv[pt, pl.ds(cs, L)] = alo | lax.shift_left(ahi, 16)
            return 0

        lax.fori_loop(0, 128 // L, ch_body, 0)
        pltpu.sync_copy(outv, out_hbm.at[pl.ds(base_pt + c * P_SUB, P_SUB)])

    gather(0, 0)
    gather(1, 1)

    def outer(c2, _):
        for slot in range(2):
            cur = c2 * 2 + slot
            pltpu.make_async_copy(ha_hbm.at[idx_v.at[cur]], rows[slot],
                                  sems[slot]).wait()
            reduce_chunk(cur, slot)

            @pl.when(cur + 2 < NSUB)
            def _():
                gather(cur + 2, slot)
        return 0

    lax.fori_loop(0, NSUB // 2, outer, 0)


def _stage_b(ha, idx3):
    mesh = plsc.VectorSubcoreMesh(core_axis_name="c", subcore_axis_name="s")
    kb = pl.kernel(
        _kb_body,
        out_type=jax.ShapeDtypeStruct((N, 128), jnp.int32),
        mesh=mesh,
        scratch_types=[
            pltpu.VMEM((NSUB, ROWS), jnp.int32),
            pltpu.VMEM((ROWS, 128), jnp.int32),
            pltpu.VMEM((ROWS, 128), jnp.int32),
            pltpu.VMEM((P_SUB, 128), jnp.int32),
            pltpu.SemaphoreType.DMA,
            pltpu.SemaphoreType.DMA,
        ],
    )
    return kb(ha, idx3)


def _kc_body(e_ref, cb_ref, w3_ref, s3_ref, c3_ref, w4_ref, s4_ref, c4_ref,
             w5_ref, b5_ref, out_ref):
    ep = e_ref[...]
    def unkey(k):
        b = jnp.where(k >= 0x8000, k - 0x8000, 0xFFFF - k).astype(jnp.uint16)
        return lax.bitcast_convert_type(b, jnp.bfloat16).astype(jnp.float32)

    vlo = unkey(ep & 0xFFFF)
    vhi = unkey(lax.shift_right_logical(ep, 16))
    e = jnp.concatenate([vlo, vhi], axis=1) + cb_ref[...]
    e = jnp.where(e >= 0, e, 0.2 * e)
    h3 = lax.dot_general(e, w3_ref[...], (((1,), (1,)), ((), ())),
                         preferred_element_type=jnp.float32) * s3_ref[...] + c3_ref[...]
    h3 = jnp.where(h3 >= 0, h3, 0.2 * h3)
    h4 = lax.dot_general(h3, w4_ref[...], (((1,), (1,)), ((), ())),
                         preferred_element_type=jnp.float32) * s4_ref[...] + c4_ref[...]
    h4 = jnp.where(h4 >= 0, h4, 0.2 * h4)
    o = lax.dot_general(w5_ref[...], h4, (((1,), (1,)), ((), ())),
                        preferred_element_type=jnp.float32) + b5_ref[...]
    out_ref[0] = o


def _stage_c(e, cb, w3, s3, c3, w4, s4, c4, w5, b5):
    grid = (B * NBLK,)
    return pl.pallas_call(
        _kc_body,
        grid=grid,
        in_specs=[
            pl.BlockSpec((BLK, 128), lambda g: (g, 0)),
            pl.BlockSpec((BLK, 256), lambda g: (g, 0)),
            pl.BlockSpec((256, 256), lambda g: (0, 0)),
            pl.BlockSpec((1, 256), lambda g: (0, 0)),
            pl.BlockSpec((1, 256), lambda g: (0, 0)),
            pl.BlockSpec((128, 256), lambda g: (0, 0)),
            pl.BlockSpec((1, 128), lambda g: (0, 0)),
            pl.BlockSpec((1, 128), lambda g: (0, 0)),
            pl.BlockSpec((13, 128), lambda g: (0, 0)),
            pl.BlockSpec((13, 1), lambda g: (0, 0)),
        ],
        out_specs=pl.BlockSpec((1, 13, BLK), lambda g: (g // NBLK, 0, g % NBLK)),
        out_shape=jax.ShapeDtypeStruct((B, 13, N), jnp.float32),
    )(e, cb, w3, s3, c3, w4, s4, c4, w5, b5)


@jax.jit
def kernel(x, p, W1, b1, g1, be1, W2, b2, g2, be2, W3, b3, g3, be3,
           W4, b4, g4, be4, W5, b5):
    rs = 1.0 / jnp.sqrt(1.0 + EPS)
    s1, s2, s3, s4 = g1 * rs, g2 * rs, g3 * rs, g4 * rs
    c1 = (b1 * s1 + be1)[None, :]
    Wa = W2[:, :256]
    Wc = W2[:, 256:] - Wa
    c2 = (b2 * s2 + be2)[None, :]
    c3 = (b3 * s3 + be3)[None, :]
    c4 = (b4 * s4 + be4)[None, :]
    b5c = b5[:, None]

    s1r, s2r, s3r, s4r = s1[None, :], s2[None, :], s3[None, :], s4[None, :]
    es = []
    cbs = []
    for b in range(B):
        ha, cb, idx = _stage_a(x, p, W1, s1r, c1, Wa, Wc, s2r, s2r, c2, b)
        cbs.append(cb)
        es.append(_stage_b(ha, idx.reshape(NW, NSUB, ROWS)))
    e = jnp.concatenate(es, axis=0)
    cbf = jnp.concatenate(cbs, axis=0)
    return _stage_c(e, cbf, W3, s3r, c3, W4, s4r, c4, W5, b5c)


# speedup vs baseline: 1.0742x; 1.0742x over previous
import jax
import jax.numpy as jnp
from jax import lax
from jax.experimental import pallas as pl
from jax.experimental.pallas import tpu as pltpu
from jax.experimental.pallas import tpu_sc as plsc

EPS = 1e-5
K = 20
B, N = 2, 2048
CIN = 1216
BLKA = 256
NBLKA = N // BLKA
BLK = 128
NBLK = N // BLK
BN = B * N

NC, NS, L = 2, 16, 16
NW = NC * NS
PTS_W = N // NW
P_SUB = 4
ROWS = P_SUB * K
NSUB = PTS_W // P_SUB


def _ka_body(x_ref, pb_ref, pf_ref, w1_ref, s1_ref, c1_ref, wa_ref, wc_ref,
             s2_ref, s2p_ref, c2_ref, ha_ref, cb_ref, idx_ref):
    xb = x_ref[0]
    h1 = lax.dot_general(xb, w1_ref[...], (((0,), (1,)), ((), ())),
                         preferred_element_type=jnp.float32)
    h1 = h1 * s1_ref[...] + c1_ref[...]
    h1 = jnp.where(h1 >= 0, h1, 0.2 * h1)
    s2 = s2_ref[...]
    ha = lax.dot_general(h1, wa_ref[...], (((1,), (1,)), ((), ())),
                         preferred_element_type=jnp.float32) * s2
    def skey(v):
        b = lax.bitcast_convert_type(v.astype(jnp.bfloat16),
                                     jnp.uint16).astype(jnp.uint32)
        return jnp.where(b >= 0x8000, 0xFFFF - b, b + 0x8000)

    lo = skey(ha[:, :128])
    hi = skey(ha[:, 128:])
    ha_ref[...] = (lo | (hi << 16)).astype(jnp.int32)
    cb_ref[...] = lax.dot_general(h1, wc_ref[...], (((1,), (1,)), ((), ())),
                                  preferred_element_type=jnp.float32) * s2p_ref[...] + c2_ref[...]

    pf = pf_ref[0]
    pb = pb_ref[0]
    ppf = jnp.sum(pf * pf, axis=0)
    ppb = jnp.sum(pb * pb, axis=0)
    G = lax.dot_general(pb, pf, (((0,), (0,)), ((), ())),
                        preferred_element_type=jnp.float32)
    nd = 2.0 * G - ppb[:, None] - ppf[None, :]
    iota_f = lax.broadcasted_iota(jnp.int32, (BLKA, N), 1).astype(jnp.float32)
    for j in range(K):
        m = jnp.max(nd, axis=1, keepdims=True)
        eqm = nd == m
        cand = jnp.where(eqm, iota_f, 4096.0)
        amin = jnp.min(cand, axis=1, keepdims=True)
        idx_ref[:, j] = amin[:, 0].astype(jnp.int32)
        nd = jnp.where(eqm, -jnp.inf, nd)


def _stage_a(x, p, w1, s1, c1, wa, wc, s2, s2p, c2, b):
    grid = (NBLKA,)
    return pl.pallas_call(
        _ka_body,
        grid=grid,
        in_specs=[
            pl.BlockSpec((1, CIN, BLKA), lambda g, _b=b: (_b, 0, g)),
            pl.BlockSpec((1, 3, BLKA), lambda g, _b=b: (_b, 0, g)),
            pl.BlockSpec((1, 3, N), lambda g, _b=b: (_b, 0, 0)),
            pl.BlockSpec((256, CIN), lambda g: (0, 0)),
            pl.BlockSpec((1, 256), lambda g: (0, 0)),
            pl.BlockSpec((1, 256), lambda g: (0, 0)),
            pl.BlockSpec((256, 256), lambda g: (0, 0)),
            pl.BlockSpec((256, 256), lambda g: (0, 0)),
            pl.BlockSpec((1, 256), lambda g: (0, 0)),
            pl.BlockSpec((1, 256), lambda g: (0, 0)),
            pl.BlockSpec((1, 256), lambda g: (0, 0)),
        ],
        out_specs=[
            pl.BlockSpec((BLKA, 128), lambda g: (g, 0)),
            pl.BlockSpec((BLKA, 256), lambda g: (g, 0)),
            pl.BlockSpec((BLKA, K), lambda g: (g, 0)),
        ],
        out_shape=[
            jax.ShapeDtypeStruct((N, 128), jnp.int32),
            jax.ShapeDtypeStruct((N, 256), jnp.float32),
            jax.ShapeDtypeStruct((N, K), jnp.int32),
        ],
    )(x, p, p, w1, s1, c1, wa, wc, s2, s2p, c2)


def _kb_body(ha_hbm, idx_hbm, out_hbm, idx_v, rows0, rows1, outv,
             sem0, sem1):
    wid = lax.axis_index("s") * NC + lax.axis_index("c")
    base_pt = wid * PTS_W
    pltpu.sync_copy(idx_hbm.at[wid], idx_v)

    rows = (rows0, rows1)
    sems = (sem0, sem1)

    def gather(c, slot):
        return pltpu.async_copy(ha_hbm.at[idx_v.at[c]], rows[slot], sems[slot])

    def reduce_chunk(c, slot):
        buf = rows[slot]

        def ch_body(ch, _):
            cs = ch * L
            for pt in range(P_SUB):
                r0 = pt * K
                v = buf[r0, pl.ds(cs, L)]
                alo = v & 0xFFFF
                ahi = lax.shift_right_logical(v, 16)
                for j in range(1, K):
                    v = buf[r0 + j, pl.ds(cs, L)]
                    alo = jnp.maximum(alo, v & 0xFFFF)
                    ahi = jnp.maximum(ahi, lax.shift_right_logical(v, 16))
                outv[pt, pl.ds(cs, L)] = alo | lax.shift_left(ahi, 16)
            return 0

        lax.fori_loop(0, 128 // L, ch_body, 0)
        pltpu.sync_copy(outv, out_hbm.at[pl.ds(base_pt + c * P_SUB, P_SUB)])

    gather(0, 0)
    gather(1, 1)

    def outer(c2, _):
        for slot in range(2):
            cur = c2 * 2 + slot
            pltpu.make_async_copy(ha_hbm.at[idx_v.at[cur]], rows[slot],
                                  sems[slot]).wait()
            reduce_chunk(cur, slot)

            @pl.when(cur + 2 < NSUB)
            def _():
                gather(cur + 2, slot)
        return 0

    lax.fori_loop(0, NSUB // 2, outer, 0)


def _stage_b(ha, idx3):
    mesh = plsc.VectorSubcoreMesh(core_axis_name="c", subcore_axis_name="s")
    kb = pl.kernel(
        _kb_body,
        out_type=jax.ShapeDtypeStruct((N, 128), jnp.int32),
        mesh=mesh,
        scratch_types=[
            pltpu.VMEM((NSUB, ROWS), jnp.int32),
            pltpu.VMEM((ROWS, 128), jnp.int32),
            pltpu.VMEM((ROWS, 128), jnp.int32),
            pltpu.VMEM((P_SUB, 128), jnp.int32),
            pltpu.SemaphoreType.DMA,
            pltpu.SemaphoreType.DMA,
        ],
    )
    return kb(ha, idx3)


def _kc_body(e_ref, cb_ref, w3_ref, s3_ref, c3_ref, w4_ref, s4_ref, c4_ref,
             w5_ref, b5_ref, out_ref):
    ep = e_ref[...]
    def unkey(k):
        b = jnp.where(k >= 0x8000, k - 0x8000, 0xFFFF - k).astype(jnp.uint16)
        return lax.bitcast_convert_type(b, jnp.bfloat16).astype(jnp.float32)

    vlo = unkey(ep & 0xFFFF)
    vhi = unkey(lax.shift_right_logical(ep, 16))
    e = jnp.concatenate([vlo, vhi], axis=1) + cb_ref[...]
    e = jnp.where(e >= 0, e, 0.2 * e)
    h3 = lax.dot_general(e, w3_ref[...], (((1,), (1,)), ((), ())),
                         preferred_element_type=jnp.float32) * s3_ref[...] + c3_ref[...]
    h3 = jnp.where(h3 >= 0, h3, 0.2 * h3)
    h4 = lax.dot_general(h3, w4_ref[...], (((1,), (1,)), ((), ())),
                         preferred_element_type=jnp.float32) * s4_ref[...] + c4_ref[...]
    h4 = jnp.where(h4 >= 0, h4, 0.2 * h4)
    o = lax.dot_general(w5_ref[...], h4, (((1,), (1,)), ((), ())),
                        preferred_element_type=jnp.float32) + b5_ref[...]
    out_ref[0] = o


def _stage_c(e, cb, w3, s3, c3, w4, s4, c4, w5, b5):
    grid = (NBLK,)
    return pl.pallas_call(
        _kc_body,
        grid=grid,
        in_specs=[
            pl.BlockSpec((BLK, 128), lambda g: (g, 0)),
            pl.BlockSpec((BLK, 256), lambda g: (g, 0)),
            pl.BlockSpec((256, 256), lambda g: (0, 0)),
            pl.BlockSpec((1, 256), lambda g: (0, 0)),
            pl.BlockSpec((1, 256), lambda g: (0, 0)),
            pl.BlockSpec((128, 256), lambda g: (0, 0)),
            pl.BlockSpec((1, 128), lambda g: (0, 0)),
            pl.BlockSpec((1, 128), lambda g: (0, 0)),
            pl.BlockSpec((13, 128), lambda g: (0, 0)),
            pl.BlockSpec((13, 1), lambda g: (0, 0)),
        ],
        out_specs=pl.BlockSpec((1, 13, BLK), lambda g: (0, 0, g)),
        out_shape=jax.ShapeDtypeStruct((1, 13, N), jnp.float32),
    )(e, cb, w3, s3, c3, w4, s4, c4, w5, b5)


@jax.jit
def kernel(x, p, W1, b1, g1, be1, W2, b2, g2, be2, W3, b3, g3, be3,
           W4, b4, g4, be4, W5, b5):
    rs = 1.0 / jnp.sqrt(1.0 + EPS)
    s1, s2, s3, s4 = g1 * rs, g2 * rs, g3 * rs, g4 * rs
    c1 = (b1 * s1 + be1)[None, :]
    Wa = W2[:, :256]
    Wc = W2[:, 256:] - Wa
    c2 = (b2 * s2 + be2)[None, :]
    c3 = (b3 * s3 + be3)[None, :]
    c4 = (b4 * s4 + be4)[None, :]
    b5c = b5[:, None]

    s1r, s2r, s3r, s4r = s1[None, :], s2[None, :], s3[None, :], s4[None, :]
    outs = []
    es = []
    cbs = []
    for b in range(B):
        ha, cb, idx = _stage_a(x, p, W1, s1r, c1, Wa, Wc, s2r, s2r, c2, b)
        cbs.append(cb)
        es.append(_stage_b(ha, idx.reshape(NW, NSUB, ROWS)))
    for b in range(B):
        outs.append(_stage_c(es[b], cbs[b], W3, s3r, c3, W4, s4r, c4, W5, b5c))
    return jnp.concatenate(outs, axis=0)


# speedup vs baseline: 1.1030x; 1.0269x over previous
import jax
import jax.numpy as jnp
from jax import lax
from jax.experimental import pallas as pl
from jax.experimental.pallas import tpu as pltpu
from jax.experimental.pallas import tpu_sc as plsc

EPS = 1e-5
K = 20
B, N = 2, 2048
CIN = 1216
BLKA = 256
NBLKA = N // BLKA
BLK = 256
NBLK = N // BLK
BN = B * N

NC, NS, L = 2, 16, 16
NW = NC * NS
PTS_W = N // NW
P_SUB = 4
ROWS = P_SUB * K
NSUB = PTS_W // P_SUB


def _ka_body(x_ref, pb_ref, pf_ref, w1_ref, s1_ref, c1_ref, wa_ref, wc_ref,
             s2_ref, s2p_ref, c2_ref, ha_ref, cb_ref, idx_ref):
    xb = x_ref[0]
    h1 = lax.dot_general(xb, w1_ref[...], (((0,), (1,)), ((), ())),
                         preferred_element_type=jnp.float32)
    h1 = h1 * s1_ref[...] + c1_ref[...]
    h1 = jnp.where(h1 >= 0, h1, 0.2 * h1)
    s2 = s2_ref[...]
    ha = lax.dot_general(h1, wa_ref[...], (((1,), (1,)), ((), ())),
                         preferred_element_type=jnp.float32) * s2
    def skey(v):
        b = lax.bitcast_convert_type(v.astype(jnp.bfloat16),
                                     jnp.uint16).astype(jnp.uint32)
        return jnp.where(b >= 0x8000, 0xFFFF - b, b + 0x8000)

    lo = skey(ha[:, :128])
    hi = skey(ha[:, 128:])
    ha_ref[...] = (lo | (hi << 16)).astype(jnp.int32)
    cb_ref[...] = lax.dot_general(h1, wc_ref[...], (((1,), (1,)), ((), ())),
                                  preferred_element_type=jnp.float32) * s2p_ref[...] + c2_ref[...]

    pf = pf_ref[0]
    pb = pb_ref[0]
    ppf = jnp.sum(pf * pf, axis=0)
    ppb = jnp.sum(pb * pb, axis=0)
    G = lax.dot_general(pb, pf, (((0,), (0,)), ((), ())),
                        preferred_element_type=jnp.float32)
    nd = 2.0 * G - ppb[:, None] - ppf[None, :]
    iota_f = lax.broadcasted_iota(jnp.int32, (BLKA, N), 1).astype(jnp.float32)
    for j in range(K):
        m = jnp.max(nd, axis=1, keepdims=True)
        eqm = nd == m
        cand = jnp.where(eqm, iota_f, 4096.0)
        amin = jnp.min(cand, axis=1, keepdims=True)
        idx_ref[:, j] = amin[:, 0].astype(jnp.int32)
        nd = jnp.where(eqm, -jnp.inf, nd)


def _stage_a(x, p, w1, s1, c1, wa, wc, s2, s2p, c2, b):
    grid = (NBLKA,)
    return pl.pallas_call(
        _ka_body,
        grid=grid,
        in_specs=[
            pl.BlockSpec((1, CIN, BLKA), lambda g, _b=b: (_b, 0, g)),
            pl.BlockSpec((1, 3, BLKA), lambda g, _b=b: (_b, 0, g)),
            pl.BlockSpec((1, 3, N), lambda g, _b=b: (_b, 0, 0)),
            pl.BlockSpec((256, CIN), lambda g: (0, 0)),
            pl.BlockSpec((1, 256), lambda g: (0, 0)),
            pl.BlockSpec((1, 256), lambda g: (0, 0)),
            pl.BlockSpec((256, 256), lambda g: (0, 0)),
            pl.BlockSpec((256, 256), lambda g: (0, 0)),
            pl.BlockSpec((1, 256), lambda g: (0, 0)),
            pl.BlockSpec((1, 256), lambda g: (0, 0)),
            pl.BlockSpec((1, 256), lambda g: (0, 0)),
        ],
        out_specs=[
            pl.BlockSpec((BLKA, 128), lambda g: (g, 0)),
            pl.BlockSpec((BLKA, 256), lambda g: (g, 0)),
            pl.BlockSpec((BLKA, K), lambda g: (g, 0)),
        ],
        out_shape=[
            jax.ShapeDtypeStruct((N, 128), jnp.int32),
            jax.ShapeDtypeStruct((N, 256), jnp.float32),
            jax.ShapeDtypeStruct((N, K), jnp.int32),
        ],
    )(x, p, p, w1, s1, c1, wa, wc, s2, s2p, c2)


def _kb_body(ha_hbm, idx_hbm, out_hbm, idx_v, rows0, rows1, outv,
             sem0, sem1):
    wid = lax.axis_index("s") * NC + lax.axis_index("c")
    base_pt = wid * PTS_W
    pltpu.sync_copy(idx_hbm.at[wid], idx_v)

    rows = (rows0, rows1)
    sems = (sem0, sem1)

    def gather(c, slot):
        return pltpu.async_copy(ha_hbm.at[idx_v.at[c]], rows[slot], sems[slot])

    def reduce_chunk(c, slot):
        buf = rows[slot]

        def ch_body(ch, _):
            cs = ch * L
            for pt in range(P_SUB):
                r0 = pt * K
                v = buf[r0, pl.ds(cs, L)]
                alo = v & 0xFFFF
                ahi = lax.shift_right_logical(v, 16)
                for j in range(1, K):
                    v = buf[r0 + j, pl.ds(cs, L)]
                    alo = jnp.maximum(alo, v & 0xFFFF)
                    ahi = jnp.maximum(ahi, lax.shift_right_logical(v, 16))
                outv[pt, pl.ds(cs, L)] = alo | lax.shift_left(ahi, 16)
            return 0

        lax.fori_loop(0, 128 // L, ch_body, 0)
        pltpu.sync_copy(outv, out_hbm.at[pl.ds(base_pt + c * P_SUB, P_SUB)])

    gather(0, 0)
    gather(1, 1)

    def outer(c2, _):
        for slot in range(2):
            cur = c2 * 2 + slot
            pltpu.make_async_copy(ha_hbm.at[idx_v.at[cur]], rows[slot],
                                  sems[slot]).wait()
            reduce_chunk(cur, slot)

            @pl.when(cur + 2 < NSUB)
            def _():
                gather(cur + 2, slot)
        return 0

    lax.fori_loop(0, NSUB // 2, outer, 0)


def _stage_b(ha, idx3):
    mesh = plsc.VectorSubcoreMesh(core_axis_name="c", subcore_axis_name="s")
    kb = pl.kernel(
        _kb_body,
        out_type=jax.ShapeDtypeStruct((N, 128), jnp.int32),
        mesh=mesh,
        scratch_types=[
            pltpu.VMEM((NSUB, ROWS), jnp.int32),
            pltpu.VMEM((ROWS, 128), jnp.int32),
            pltpu.VMEM((ROWS, 128), jnp.int32),
            pltpu.VMEM((P_SUB, 128), jnp.int32),
            pltpu.SemaphoreType.DMA,
            pltpu.SemaphoreType.DMA,
        ],
    )
    return kb(ha, idx3)


def _kc_body(e_ref, cb_ref, w3_ref, s3_ref, c3_ref, w4_ref, s4_ref, c4_ref,
             w5_ref, b5_ref, out_ref):
    ep = e_ref[...]
    def unkey(k):
        b = jnp.where(k >= 0x8000, k - 0x8000, 0xFFFF - k).astype(jnp.uint16)
        return lax.bitcast_convert_type(b, jnp.bfloat16).astype(jnp.float32)

    vlo = unkey(ep & 0xFFFF)
    vhi = unkey(lax.shift_right_logical(ep, 16))
    e = jnp.concatenate([vlo, vhi], axis=1) + cb_ref[...]
    e = jnp.where(e >= 0, e, 0.2 * e)
    h3 = lax.dot_general(e, w3_ref[...], (((1,), (1,)), ((), ())),
                         preferred_element_type=jnp.float32) * s3_ref[...] + c3_ref[...]
    h3 = jnp.where(h3 >= 0, h3, 0.2 * h3)
    h4 = lax.dot_general(h3, w4_ref[...], (((1,), (1,)), ((), ())),
                         preferred_element_type=jnp.float32) * s4_ref[...] + c4_ref[...]
    h4 = jnp.where(h4 >= 0, h4, 0.2 * h4)
    o = lax.dot_general(w5_ref[...], h4, (((1,), (1,)), ((), ())),
                        preferred_element_type=jnp.float32) + b5_ref[...]
    out_ref[0] = o


def _stage_c(e, cb, w3, s3, c3, w4, s4, c4, w5, b5):
    grid = (NBLK,)
    return pl.pallas_call(
        _kc_body,
        grid=grid,
        in_specs=[
            pl.BlockSpec((BLK, 128), lambda g: (g, 0)),
            pl.BlockSpec((BLK, 256), lambda g: (g, 0)),
            pl.BlockSpec((256, 256), lambda g: (0, 0)),
            pl.BlockSpec((1, 256), lambda g: (0, 0)),
            pl.BlockSpec((1, 256), lambda g: (0, 0)),
            pl.BlockSpec((128, 256), lambda g: (0, 0)),
            pl.BlockSpec((1, 128), lambda g: (0, 0)),
            pl.BlockSpec((1, 128), lambda g: (0, 0)),
            pl.BlockSpec((13, 128), lambda g: (0, 0)),
            pl.BlockSpec((13, 1), lambda g: (0, 0)),
        ],
        out_specs=pl.BlockSpec((1, 13, BLK), lambda g: (0, 0, g)),
        out_shape=jax.ShapeDtypeStruct((1, 13, N), jnp.float32),
    )(e, cb, w3, s3, c3, w4, s4, c4, w5, b5)


@jax.jit
def kernel(x, p, W1, b1, g1, be1, W2, b2, g2, be2, W3, b3, g3, be3,
           W4, b4, g4, be4, W5, b5):
    rs = 1.0 / jnp.sqrt(1.0 + EPS)
    s1, s2, s3, s4 = g1 * rs, g2 * rs, g3 * rs, g4 * rs
    c1 = (b1 * s1 + be1)[None, :]
    Wa = W2[:, :256]
    Wc = W2[:, 256:] - Wa
    c2 = (b2 * s2 + be2)[None, :]
    c3 = (b3 * s3 + be3)[None, :]
    c4 = (b4 * s4 + be4)[None, :]
    b5c = b5[:, None]

    s1r, s2r, s3r, s4r = s1[None, :], s2[None, :], s3[None, :], s4[None, :]
    outs = []
    es = []
    cbs = []
    for b in range(B):
        ha, cb, idx = _stage_a(x, p, W1, s1r, c1, Wa, Wc, s2r, s2r, c2, b)
        cbs.append(cb)
        es.append(_stage_b(ha, idx.reshape(NW, NSUB, ROWS)))
    for b in range(B):
        outs.append(_stage_c(es[b], cbs[b], W3, s3r, c3, W4, s4r, c4, W5, b5c))
    return jnp.concatenate(outs, axis=0)
